# baseline (speedup 1.0000x reference)
import functools

import jax
import jax.numpy as jnp
from jax import lax
from jax.experimental import pallas as pl
from jax.experimental.pallas import tpu as pltpu
from jax.experimental.pallas import tpu_sc as plsc

N = 10000
E = 320000
F = 128
G = 64
ITERS = 10

NC = 2
NS = 16
NW = NC * NS
CHUNK = 96
NBUF = 3
HALVES = 4
CHUNKS_PER_W = 108
CH_HALF = CHUNKS_PER_W // HALVES
NGROUPS = CH_HALF // NBUF
E_PAD = NW * CHUNKS_PER_W * CHUNK
ACC_ROWS = 10240
ROWS_PER_TILE = ACC_ROWS // NS


def _sc_aggregate(src_chunks, dst_chunks, x, zeros_blk):
    mesh = plsc.VectorSubcoreMesh(core_axis_name="c", subcore_axis_name="s")

    @functools.partial(
        pl.kernel,
        out_type=jax.ShapeDtypeStruct((NC, ACC_ROWS, F), jnp.float32),
        mesh=mesh,
        scratch_types=[
            pltpu.VMEM((CH_HALF, CHUNK), jnp.int32),
            pltpu.VMEM((CH_HALF, CHUNK), jnp.int32),
            pltpu.VMEM((NBUF, CHUNK, F), jnp.float32),
            pltpu.VMEM_SHARED((ACC_ROWS, F), jnp.float32),
        ] + [pltpu.SemaphoreType.DMA] * (2 * NBUF),
    )
    def agg_kernel(src_hbm, dst_hbm, x_hbm, zeros_hbm, out_hbm,
                   sidx_v, didx_v, bufs, acc, *sems):
        gsems = sems[:NBUF]
        ssems = sems[NBUF:]
        cid = lax.axis_index("c")
        sid = lax.axis_index("s")
        wid = cid * NS + sid
        base = sid * ROWS_PER_TILE

        pltpu.sync_copy(zeros_hbm, bufs.at[0])
        for t in range(ROWS_PER_TILE // CHUNK):
            pltpu.sync_copy(bufs.at[0], acc.at[pl.ds(base + t * CHUNK, CHUNK)])
        REM = ROWS_PER_TILE % CHUNK
        if REM:
            pltpu.sync_copy(
                bufs.at[0, pl.ds(0, REM)],
                acc.at[pl.ds(base + ROWS_PER_TILE - REM, REM)])
        plsc.subcore_barrier()

        for half in range(HALVES):
            pltpu.sync_copy(src_hbm.at[wid, half], sidx_v)
            pltpu.sync_copy(dst_hbm.at[wid, half], didx_v)

            for b in range(NBUF):
                pltpu.async_copy(x_hbm.at[sidx_v.at[b]], bufs.at[b], gsems[b])

            def group(g, carry):
                j0 = g * NBUF
                for b in range(NBUF):
                    pltpu.make_async_copy(
                        x_hbm.at[sidx_v.at[j0 + b]], bufs.at[b],
                        gsems[b]).wait()
                    pltpu.sync_copy(bufs.at[b], acc.at[didx_v.at[j0 + b]],
                                    add=True)
                    nj = jnp.minimum(j0 + NBUF + b, CH_HALF - 1)
                    pltpu.async_copy(
                        x_hbm.at[sidx_v.at[nj]], bufs.at[b], gsems[b])
                return carry

            lax.fori_loop(0, NGROUPS, group, 0)
            for b in range(NBUF):
                pltpu.make_async_copy(
                    x_hbm.at[sidx_v.at[CH_HALF - 1]], bufs.at[b],
                    gsems[b]).wait()

        plsc.subcore_barrier()

        for t in range(ROWS_PER_TILE // CHUNK):
            r0 = base + t * CHUNK
            pltpu.sync_copy(acc.at[pl.ds(r0, CHUNK)], bufs.at[0])
            pltpu.sync_copy(bufs.at[0], out_hbm.at[cid, pl.ds(r0, CHUNK)])
        if REM:
            r0 = base + ROWS_PER_TILE - REM
            pltpu.sync_copy(acc.at[pl.ds(r0, REM)], bufs.at[0, pl.ds(0, REM)])
            pltpu.sync_copy(bufs.at[0, pl.ds(0, REM)],
                            out_hbm.at[cid, pl.ds(r0, REM)])

    return agg_kernel(src_chunks, dst_chunks, x, zeros_blk)


BN = 1000
NBLK = N // BN


def _dense_kernel(x_ref, p_ref, b_ref, w1a_ref, b1a_ref, w1b_ref, b1b_ref,
                  gamma_ref, beta_ref, wtop_ref, wbot_ref, bcat_ref,
                  wm1_ref, bm1_ref, wm2_ref, bm2_ref, out_ref,
                  sum_s, sq_s, seg_s, cnt_s):
    i = pl.program_id(0)

    @pl.when(i == 0)
    def _init():
        sum_s[...] = jnp.zeros_like(sum_s)
        sq_s[...] = jnp.zeros_like(sq_s)
        seg_s[...] = jnp.zeros_like(seg_s)
        cnt_s[...] = jnp.zeros_like(cnt_s)

    h = x_ref[...] + p_ref[0] + p_ref[1]
    h = lax.dot_general(h, w1a_ref[...], (((1,), (0,)), ((), ())),
                        preferred_element_type=jnp.float32) + b1a_ref[...]
    h = jnp.maximum(h, 0.0)
    h = lax.dot_general(h, w1b_ref[...], (((1,), (0,)), ((), ())),
                        preferred_element_type=jnp.float32) + b1b_ref[...]
    x1 = jnp.maximum(h, 0.0)

    sum_s[...] += jnp.sum(x1, axis=0, keepdims=True)
    sq_s[...] += jnp.sum(x1 * x1, axis=0, keepdims=True)

    bb = b_ref[0]
    onehot = (bb.reshape(BN, 1) ==
              lax.broadcasted_iota(jnp.int32, (1, G), 1)).astype(jnp.float32)
    seg_s[...] += lax.dot_general(onehot, x1, (((0,), (0,)), ((), ())),
                                  preferred_element_type=jnp.float32)
    cnt_s[...] += jnp.sum(onehot, axis=0, keepdims=True)

    @pl.when(i == NBLK - 1)
    def _finish():
        mean = sum_s[...] / float(N)
        var = sq_s[...] / float(N) - mean * mean
        scale = gamma_ref[...] * lax.rsqrt(var + 1e-5)
        counts = cnt_s[...]
        counts_col = counts.reshape(G, 1)
        seg = seg_s[...]
        pooled = scale * (seg - counts_col * mean) + counts_col * beta_ref[...]
        x_new = pooled / jnp.maximum(counts_col, 1.0)

        base = lax.dot_general(x_new, wtop_ref[...], (((1,), (0,)), ((), ())),
                               preferred_element_type=jnp.float32) + bcat_ref[...]
        hh = x_new
        for _ in range(ITERS):
            pq = base + lax.dot_general(hh, wbot_ref[...],
                                        (((1,), (0,)), ((), ())),
                                        preferred_element_type=jnp.float32)
            p = pq[:, :G]
            q = pq[:, G:]
            hh = jnp.concatenate(
                [jnp.tanh(q), 1.0 / (1.0 + jnp.exp(-p))], axis=1)

        o = lax.dot_general(hh, wm1_ref[...], (((1,), (0,)), ((), ())),
                            preferred_element_type=jnp.float32) + bm1_ref[...]
        o = jnp.maximum(o, 0.0)
        o = lax.dot_general(o, wm2_ref[...], (((1,), (0,)), ((), ())),
                            preferred_element_type=jnp.float32) + bm2_ref[...]
        out_ref[...] = o


def kernel(x, edge_index, batch, W1a, b1a, W1b, b1b, gamma, beta,
           Wl1, bl1, Wl2, bl2, Wm1, bm1, Wm2, bm2):
    src = edge_index[0]
    dst = edge_index[1]
    epw = E // NW
    ppw = CHUNKS_PER_W * CHUNK - epw
    pad_ids = (jnp.arange(ppw, dtype=jnp.int32) % (ACC_ROWS - N))[None]
    src_p = jnp.concatenate(
        [src.reshape(NW, epw), jnp.broadcast_to(pad_ids, (NW, ppw))], axis=1)
    dst_p = jnp.concatenate(
        [dst.reshape(NW, epw), jnp.broadcast_to(N + pad_ids, (NW, ppw))],
        axis=1)
    src_chunks = src_p.reshape(NW, HALVES, CH_HALF, CHUNK)
    dst_chunks = dst_p.reshape(NW, HALVES, CH_HALF, CHUNK)
    zeros_blk = jnp.zeros((CHUNK, F), jnp.float32)

    partials = _sc_aggregate(src_chunks, dst_chunks, x, zeros_blk)

    wtop = jnp.concatenate([Wl1[:F], Wl2[:F]], axis=1)
    wbot = jnp.concatenate([Wl1[F:], Wl2[F:]], axis=1)
    bcat = jnp.concatenate([bl1, bl2]).reshape(1, 2 * (F // 2))
    DIM = Wm1.shape[1]
    OUT = Wm2.shape[1]
    wm1p = jnp.zeros((F, F), jnp.float32).at[:, :DIM].set(Wm1)
    bm1p = jnp.zeros((1, F), jnp.float32).at[0, :DIM].set(bm1)
    wm2p = jnp.zeros((F, F), jnp.float32).at[:DIM, :OUT].set(Wm2)
    bm2p = jnp.zeros((1, F), jnp.float32).at[0, :OUT].set(bm2)
    batch3 = batch.reshape(NBLK, 1, BN)

    row_spec = pl.BlockSpec((BN, F), lambda i: (i, 0))
    full = lambda shape: pl.BlockSpec(shape, lambda i: tuple(0 for _ in shape))

    out = pl.pallas_call(
        _dense_kernel,
        grid=(NBLK,),
        in_specs=[
            row_spec,
            pl.BlockSpec((NC, BN, F), lambda i: (0, i, 0)),
            pl.BlockSpec((1, 1, BN), lambda i: (i, 0, 0)),
            full((F, F)), full((1, F)), full((F, F)), full((1, F)),
            full((1, F)), full((1, F)),
            full((F, F)), full((F, F)), full((1, F)),
            full((F, F)), full((1, F)), full((F, F)), full((1, F)),
        ],
        out_specs=pl.BlockSpec((G, F), lambda i: (0, 0)),
        out_shape=jax.ShapeDtypeStruct((G, F), jnp.float32),
        scratch_shapes=[
            pltpu.VMEM((1, F), jnp.float32),
            pltpu.VMEM((1, F), jnp.float32),
            pltpu.VMEM((G, F), jnp.float32),
            pltpu.VMEM((1, G), jnp.float32),
        ],
    )(x, partials, batch3, W1a, b1a.reshape(1, F), W1b, b1b.reshape(1, F),
      gamma.reshape(1, F), beta.reshape(1, F), wtop, wbot, bcat,
      wm1p, bm1p, wm2p, bm2p)

    return out[:, :OUT]


# speedup vs baseline: 1.0310x; 1.0310x over previous
import functools

import jax
import jax.numpy as jnp
from jax import lax
from jax.experimental import pallas as pl
from jax.experimental.pallas import tpu as pltpu
from jax.experimental.pallas import tpu_sc as plsc

N = 10000
E = 320000
F = 128
G = 64
ITERS = 10

NC = 2
NS = 16
NW = NC * NS
CHUNK = 128
NBUF = 2
HALVES = 2
CHUNKS_PER_W = 80
CH_HALF = CHUNKS_PER_W // HALVES
NGROUPS = CH_HALF // NBUF
E_PAD = NW * CHUNKS_PER_W * CHUNK
ACC_ROWS = 10240
ROWS_PER_TILE = ACC_ROWS // NS


def _sc_aggregate(src_chunks, dst_chunks, x, zeros_blk):
    mesh = plsc.VectorSubcoreMesh(core_axis_name="c", subcore_axis_name="s")

    @functools.partial(
        pl.kernel,
        out_type=jax.ShapeDtypeStruct((NC, ACC_ROWS, F), jnp.float32),
        mesh=mesh,
        scratch_types=[
            pltpu.VMEM((CH_HALF, CHUNK), jnp.int32),
            pltpu.VMEM((CH_HALF, CHUNK), jnp.int32),
            pltpu.VMEM((NBUF, CHUNK, F), jnp.float32),
            pltpu.VMEM_SHARED((ACC_ROWS, F), jnp.float32),
        ] + [pltpu.SemaphoreType.DMA] * (2 * NBUF),
    )
    def agg_kernel(src_hbm, dst_hbm, x_hbm, zeros_hbm, out_hbm,
                   sidx_v, didx_v, bufs, acc, *sems):
        gsems = sems[:NBUF]
        ssems = sems[NBUF:]
        cid = lax.axis_index("c")
        sid = lax.axis_index("s")
        wid = cid * NS + sid
        base = sid * ROWS_PER_TILE

        pltpu.sync_copy(zeros_hbm, bufs.at[0])
        for t in range(ROWS_PER_TILE // CHUNK):
            pltpu.sync_copy(bufs.at[0], acc.at[pl.ds(base + t * CHUNK, CHUNK)])
        REM = ROWS_PER_TILE % CHUNK
        if REM:
            pltpu.sync_copy(
                bufs.at[0, pl.ds(0, REM)],
                acc.at[pl.ds(base + ROWS_PER_TILE - REM, REM)])
        plsc.subcore_barrier()

        for half in range(HALVES):
            pltpu.sync_copy(src_hbm.at[wid, half], sidx_v)
            pltpu.sync_copy(dst_hbm.at[wid, half], didx_v)

            for b in range(NBUF):
                pltpu.async_copy(x_hbm.at[sidx_v.at[b]], bufs.at[b], gsems[b])

            def group(g, carry):
                j0 = g * NBUF
                for b in range(NBUF):
                    pltpu.make_async_copy(
                        x_hbm.at[sidx_v.at[j0 + b]], bufs.at[b],
                        gsems[b]).wait()
                    pltpu.sync_copy(bufs.at[b], acc.at[didx_v.at[j0 + b]],
                                    add=True)
                    nj = jnp.minimum(j0 + NBUF + b, CH_HALF - 1)
                    pltpu.async_copy(
                        x_hbm.at[sidx_v.at[nj]], bufs.at[b], gsems[b])
                return carry

            lax.fori_loop(0, NGROUPS, group, 0)
            for b in range(NBUF):
                pltpu.make_async_copy(
                    x_hbm.at[sidx_v.at[CH_HALF - 1]], bufs.at[b],
                    gsems[b]).wait()

        plsc.subcore_barrier()

        for t in range(ROWS_PER_TILE // CHUNK):
            r0 = base + t * CHUNK
            pltpu.sync_copy(acc.at[pl.ds(r0, CHUNK)], bufs.at[0])
            pltpu.sync_copy(bufs.at[0], out_hbm.at[cid, pl.ds(r0, CHUNK)])
        if REM:
            r0 = base + ROWS_PER_TILE - REM
            pltpu.sync_copy(acc.at[pl.ds(r0, REM)], bufs.at[0, pl.ds(0, REM)])
            pltpu.sync_copy(bufs.at[0, pl.ds(0, REM)],
                            out_hbm.at[cid, pl.ds(r0, REM)])

    return agg_kernel(src_chunks, dst_chunks, x, zeros_blk)


BN = 1000
NBLK = N // BN


def _dense_kernel(x_ref, p_ref, b_ref, w1a_ref, b1a_ref, w1b_ref, b1b_ref,
                  gamma_ref, beta_ref, wl1_ref, bl1_ref, wl2_ref, bl2_ref,
                  wm1_ref, bm1_ref, wm2_ref, bm2_ref, out_ref,
                  sum_s, sq_s, seg_s, cnt_s):
    i = pl.program_id(0)

    @pl.when(i == 0)
    def _init():
        sum_s[...] = jnp.zeros_like(sum_s)
        sq_s[...] = jnp.zeros_like(sq_s)
        seg_s[...] = jnp.zeros_like(seg_s)
        cnt_s[...] = jnp.zeros_like(cnt_s)

    h = x_ref[...] + p_ref[0] + p_ref[1]
    h = lax.dot_general(h, w1a_ref[...], (((1,), (0,)), ((), ())),
                        preferred_element_type=jnp.float32) + b1a_ref[...]
    h = jnp.maximum(h, 0.0)
    h = lax.dot_general(h, w1b_ref[...], (((1,), (0,)), ((), ())),
                        preferred_element_type=jnp.float32) + b1b_ref[...]
    x1 = jnp.maximum(h, 0.0)

    sum_s[...] += jnp.sum(x1, axis=0, keepdims=True)
    sq_s[...] += jnp.sum(x1 * x1, axis=0, keepdims=True)

    bb = b_ref[0]
    onehot = (bb.reshape(BN, 1) ==
              lax.broadcasted_iota(jnp.int32, (1, G), 1)).astype(jnp.float32)
    seg_s[...] += lax.dot_general(onehot, x1, (((0,), (0,)), ((), ())),
                                  preferred_element_type=jnp.float32)
    cnt_s[...] += jnp.sum(onehot, axis=0, keepdims=True)

    @pl.when(i == NBLK - 1)
    def _finish():
        mean = sum_s[...] / float(N)
        var = sq_s[...] / float(N) - mean * mean
        scale = gamma_ref[...] * lax.rsqrt(var + 1e-5)
        counts = cnt_s[...]
        counts_col = counts.reshape(G, 1)
        seg = seg_s[...]
        pooled = scale * (seg - counts_col * mean) + counts_col * beta_ref[...]
        x_new = pooled / jnp.maximum(counts_col, 1.0)

        def mm(a, b):
            return lax.dot_general(a, b, (((1,), (0,)), ((), ())),
                                   preferred_element_type=jnp.float32)

        wl1 = wl1_ref[...]
        wl2 = wl2_ref[...]
        base_p = mm(x_new, wl1[:F]) + bl1_ref[...]
        base_q = mm(x_new, wl2[:F]) + bl2_ref[...]
        hh = x_new
        for _ in range(ITERS):
            p = base_p + mm(hh, wl1[F:])
            q = base_q + mm(hh, wl2[F:])
            hh = jnp.concatenate(
                [jnp.tanh(q), 1.0 / (1.0 + jnp.exp(-p))], axis=1)

        o = jnp.maximum(mm(hh, wm1_ref[...]) + bm1_ref[...], 0.0)
        o = mm(o, wm2_ref[...]) + bm2_ref[...]
        out_ref[...] = o


def kernel(x, edge_index, batch, W1a, b1a, W1b, b1b, gamma, beta,
           Wl1, bl1, Wl2, bl2, Wm1, bm1, Wm2, bm2):
    src = edge_index[0]
    dst = edge_index[1]
    epw = E // NW
    ppw = CHUNKS_PER_W * CHUNK - epw
    pad_ids = (jnp.arange(ppw, dtype=jnp.int32) % (ACC_ROWS - N))[None]
    src_p = jnp.concatenate(
        [src.reshape(NW, epw), jnp.broadcast_to(pad_ids, (NW, ppw))], axis=1)
    dst_p = jnp.concatenate(
        [dst.reshape(NW, epw), jnp.broadcast_to(N + pad_ids, (NW, ppw))],
        axis=1)
    src_chunks = src_p.reshape(NW, HALVES, CH_HALF, CHUNK)
    dst_chunks = dst_p.reshape(NW, HALVES, CH_HALF, CHUNK)
    zeros_blk = jnp.zeros((CHUNK, F), jnp.float32)

    partials = _sc_aggregate(src_chunks, dst_chunks, x, zeros_blk)

    DIM = Wm1.shape[1]
    OUT = Wm2.shape[1]
    batch3 = batch.reshape(NBLK, 1, BN)

    row_spec = pl.BlockSpec((BN, F), lambda i: (i, 0))
    full = lambda shape: pl.BlockSpec(shape, lambda i: tuple(0 for _ in shape))

    out = pl.pallas_call(
        _dense_kernel,
        grid=(NBLK,),
        in_specs=[
            row_spec,
            pl.BlockSpec((NC, BN, F), lambda i: (0, i, 0)),
            pl.BlockSpec((1, 1, BN), lambda i: (i, 0, 0)),
            full((F, F)), full((1, F)), full((F, F)), full((1, F)),
            full((1, F)), full((1, F)),
            full((2 * F, G)), full((1, G)), full((2 * F, G)), full((1, G)),
            full((F, DIM)), full((1, DIM)), full((DIM, OUT)), full((1, OUT)),
        ],
        out_specs=pl.BlockSpec((G, OUT), lambda i: (0, 0)),
        out_shape=jax.ShapeDtypeStruct((G, OUT), jnp.float32),
        scratch_shapes=[
            pltpu.VMEM((1, F), jnp.float32),
            pltpu.VMEM((1, F), jnp.float32),
            pltpu.VMEM((G, F), jnp.float32),
            pltpu.VMEM((1, G), jnp.float32),
        ],
    )(x, partials, batch3, W1a, b1a.reshape(1, F), W1b, b1b.reshape(1, F),
      gamma.reshape(1, F), beta.reshape(1, F),
      Wl1, bl1.reshape(1, G), Wl2, bl2.reshape(1, G),
      Wm1, bm1.reshape(1, DIM), Wm2, bm2.reshape(1, OUT))

    return out


# speedup vs baseline: 1.0566x; 1.0249x over previous
import functools

import jax
import jax.numpy as jnp
from jax import lax
from jax.experimental import pallas as pl
from jax.experimental.pallas import tpu as pltpu
from jax.experimental.pallas import tpu_sc as plsc

N = 10000
E = 320000
F = 128
G = 64
ITERS = 10

NC = 2
NS = 16
NW = NC * NS
CHUNK = 128
NBUF = 2
NCHUNKS = E // CHUNK
CH80 = 80
CH72 = 72
W72 = 8
NEXTRA = NCHUNKS - W72 * CH72 - (NW - W72) * CH80
STAGE = 40
ACC_ROWS = 10240
ROWS_PER_TILE = ACC_ROWS // NS


def _sc_aggregate(src_chunks, dst_chunks, x, zeros_blk):
    mesh = plsc.VectorSubcoreMesh(core_axis_name="c", subcore_axis_name="s")

    @functools.partial(
        pl.kernel,
        out_type=jax.ShapeDtypeStruct((NC, ACC_ROWS, F), jnp.float32),
        mesh=mesh,
        scratch_types=[
            pltpu.VMEM((STAGE, CHUNK), jnp.int32),
            pltpu.VMEM((STAGE, CHUNK), jnp.int32),
            pltpu.VMEM((NBUF, CHUNK, F), jnp.float32),
            pltpu.VMEM_SHARED((ACC_ROWS, F), jnp.float32),
        ] + [pltpu.SemaphoreType.DMA] * (2 * NBUF),
    )
    def agg_kernel(src_hbm, dst_hbm, x_hbm, zeros_hbm, out_hbm,
                   sidx_v, didx_v, bufs, acc, *sems):
        gsems = sems[:NBUF]
        ssems = sems[NBUF:]
        cid = lax.axis_index("c")
        sid = lax.axis_index("s")
        wid = cid * NS + sid
        base = sid * ROWS_PER_TILE

        pltpu.sync_copy(zeros_hbm, bufs.at[0])
        for t in range(ROWS_PER_TILE // CHUNK):
            pltpu.sync_copy(bufs.at[0], acc.at[pl.ds(base + t * CHUNK, CHUNK)])
        REM = ROWS_PER_TILE % CHUNK
        if REM:
            pltpu.sync_copy(
                bufs.at[0, pl.ds(0, REM)],
                acc.at[pl.ds(base + ROWS_PER_TILE - REM, REM)])
        plsc.subcore_barrier()

        c0 = jnp.where(wid < W72, CH72 * wid,
                       W72 * CH72 + CH80 * (wid - W72))
        nch2 = jnp.where(wid < W72, CH72 - STAGE, CH80 - STAGE)

        def run_stage(row0, ngroups, last_idx):
            row0 = pl.multiple_of(row0, 8)
            pltpu.sync_copy(src_hbm.at[pl.ds(row0, STAGE)], sidx_v)
            pltpu.sync_copy(dst_hbm.at[pl.ds(row0, STAGE)], didx_v)

            for b in range(NBUF):
                pltpu.async_copy(x_hbm.at[sidx_v.at[b]], bufs.at[b], gsems[b])

            def group(g, carry):
                j0 = g * NBUF
                for b in range(NBUF):
                    pltpu.make_async_copy(
                        x_hbm.at[sidx_v.at[j0 + b]], bufs.at[b],
                        gsems[b]).wait()
                    pltpu.sync_copy(bufs.at[b], acc.at[didx_v.at[j0 + b]],
                                    add=True)
                    nj = jnp.minimum(j0 + NBUF + b, last_idx)
                    pltpu.async_copy(
                        x_hbm.at[sidx_v.at[nj]], bufs.at[b], gsems[b])
                return carry

            lax.fori_loop(0, ngroups, group, 0)
            for b in range(NBUF):
                pltpu.make_async_copy(
                    x_hbm.at[sidx_v.at[last_idx]], bufs.at[b],
                    gsems[b]).wait()

        run_stage(c0, STAGE // NBUF, STAGE - 1)
        run_stage(c0 + STAGE, nch2 // NBUF, nch2 - 1)

        @pl.when(wid < NEXTRA)
        def _extra():
            r0 = W72 * CH72 + (NW - W72) * CH80
            pltpu.sync_copy(src_hbm.at[pl.ds(r0, NEXTRA)],
                            sidx_v.at[pl.ds(0, NEXTRA)])
            pltpu.sync_copy(dst_hbm.at[pl.ds(r0, NEXTRA)],
                            didx_v.at[pl.ds(0, NEXTRA)])
            pltpu.async_copy(
                x_hbm.at[sidx_v.at[wid]], bufs.at[0], gsems[0]).wait()
            pltpu.sync_copy(bufs.at[0], acc.at[didx_v.at[wid]], add=True)

        plsc.subcore_barrier()

        for t in range(ROWS_PER_TILE // CHUNK):
            r0 = base + t * CHUNK
            pltpu.sync_copy(acc.at[pl.ds(r0, CHUNK)], bufs.at[0])
            pltpu.sync_copy(bufs.at[0], out_hbm.at[cid, pl.ds(r0, CHUNK)])
        if REM:
            r0 = base + ROWS_PER_TILE - REM
            pltpu.sync_copy(acc.at[pl.ds(r0, REM)], bufs.at[0, pl.ds(0, REM)])
            pltpu.sync_copy(bufs.at[0, pl.ds(0, REM)],
                            out_hbm.at[cid, pl.ds(r0, REM)])

    return agg_kernel(src_chunks, dst_chunks, x, zeros_blk)


BN = 1000
NBLK = N // BN


def _dense_kernel(x_ref, p_ref, b_ref, w1a_ref, b1a_ref, w1b_ref, b1b_ref,
                  gamma_ref, beta_ref, wl1_ref, bl1_ref, wl2_ref, bl2_ref,
                  wm1_ref, bm1_ref, wm2_ref, bm2_ref, out_ref,
                  sum_s, sq_s, seg_s, cnt_s):
    i = pl.program_id(0)

    @pl.when(i == 0)
    def _init():
        sum_s[...] = jnp.zeros_like(sum_s)
        sq_s[...] = jnp.zeros_like(sq_s)
        seg_s[...] = jnp.zeros_like(seg_s)
        cnt_s[...] = jnp.zeros_like(cnt_s)

    h = x_ref[...] + p_ref[0] + p_ref[1]
    h = lax.dot_general(h, w1a_ref[...], (((1,), (0,)), ((), ())),
                        preferred_element_type=jnp.float32) + b1a_ref[...]
    h = jnp.maximum(h, 0.0)
    h = lax.dot_general(h, w1b_ref[...], (((1,), (0,)), ((), ())),
                        preferred_element_type=jnp.float32) + b1b_ref[...]
    x1 = jnp.maximum(h, 0.0)

    sum_s[...] += jnp.sum(x1, axis=0, keepdims=True)
    sq_s[...] += jnp.sum(x1 * x1, axis=0, keepdims=True)

    bb = b_ref[0]
    onehot = (bb.reshape(BN, 1) ==
              lax.broadcasted_iota(jnp.int32, (1, G), 1)).astype(jnp.float32)
    seg_s[...] += lax.dot_general(onehot, x1, (((0,), (0,)), ((), ())),
                                  preferred_element_type=jnp.float32)
    cnt_s[...] += jnp.sum(onehot, axis=0, keepdims=True)

    @pl.when(i == NBLK - 1)
    def _finish():
        mean = sum_s[...] / float(N)
        var = sq_s[...] / float(N) - mean * mean
        scale = gamma_ref[...] * lax.rsqrt(var + 1e-5)
        counts = cnt_s[...]
        counts_col = counts.reshape(G, 1)
        seg = seg_s[...]
        pooled = scale * (seg - counts_col * mean) + counts_col * beta_ref[...]
        x_new = pooled / jnp.maximum(counts_col, 1.0)

        def mm(a, b):
            return lax.dot_general(a, b, (((1,), (0,)), ((), ())),
                                   preferred_element_type=jnp.float32)

        wl1 = wl1_ref[...]
        wl2 = wl2_ref[...]
        base_p = mm(x_new, wl1[:F]) + bl1_ref[...]
        base_q = mm(x_new, wl2[:F]) + bl2_ref[...]
        hh = x_new
        for _ in range(ITERS):
            p = base_p + mm(hh, wl1[F:])
            q = base_q + mm(hh, wl2[F:])
            hh = jnp.concatenate(
                [jnp.tanh(q), 1.0 / (1.0 + jnp.exp(-p))], axis=1)

        o = jnp.maximum(mm(hh, wm1_ref[...]) + bm1_ref[...], 0.0)
        o = mm(o, wm2_ref[...]) + bm2_ref[...]
        out_ref[...] = o


def kernel(x, edge_index, batch, W1a, b1a, W1b, b1b, gamma, beta,
           Wl1, bl1, Wl2, bl2, Wm1, bm1, Wm2, bm2):
    src_chunks = edge_index[0].reshape(NCHUNKS, CHUNK)
    dst_chunks = edge_index[1].reshape(NCHUNKS, CHUNK)
    zeros_blk = jnp.zeros((CHUNK, F), jnp.float32)

    partials = _sc_aggregate(src_chunks, dst_chunks, x, zeros_blk)

    DIM = Wm1.shape[1]
    OUT = Wm2.shape[1]
    batch3 = batch.reshape(NBLK, 1, BN)

    row_spec = pl.BlockSpec((BN, F), lambda i: (i, 0))
    full = lambda shape: pl.BlockSpec(shape, lambda i: tuple(0 for _ in shape))

    out = pl.pallas_call(
        _dense_kernel,
        grid=(NBLK,),
        in_specs=[
            row_spec,
            pl.BlockSpec((NC, BN, F), lambda i: (0, i, 0)),
            pl.BlockSpec((1, 1, BN), lambda i: (i, 0, 0)),
            full((F, F)), full((1, F)), full((F, F)), full((1, F)),
            full((1, F)), full((1, F)),
            full((2 * F, G)), full((1, G)), full((2 * F, G)), full((1, G)),
            full((F, DIM)), full((1, DIM)), full((DIM, OUT)), full((1, OUT)),
        ],
        out_specs=pl.BlockSpec((G, OUT), lambda i: (0, 0)),
        out_shape=jax.ShapeDtypeStruct((G, OUT), jnp.float32),
        scratch_shapes=[
            pltpu.VMEM((1, F), jnp.float32),
            pltpu.VMEM((1, F), jnp.float32),
            pltpu.VMEM((G, F), jnp.float32),
            pltpu.VMEM((1, G), jnp.float32),
        ],
    )(x, partials, batch3, W1a, b1a.reshape(1, F), W1b, b1b.reshape(1, F),
      gamma.reshape(1, F), beta.reshape(1, F),
      Wl1, bl1.reshape(1, G), Wl2, bl2.reshape(1, G),
      Wm1, bm1.reshape(1, DIM), Wm2, bm2.reshape(1, OUT))

    return out


# speedup vs baseline: 1.1763x; 1.1132x over previous
import functools

import jax
import jax.numpy as jnp
from jax import lax
from jax.experimental import pallas as pl
from jax.experimental.pallas import tpu as pltpu
from jax.experimental.pallas import tpu_sc as plsc

N = 10000
E = 320000
F = 128
G = 64
ITERS = 10

NC = 2
NS = 16
NW = NC * NS
CHUNK = 128
NBUF = 2
NCHUNKS = E // CHUNK
CH80 = 80
CH72 = 72
W72 = 8
NEXTRA = NCHUNKS - W72 * CH72 - (NW - W72) * CH80
STAGE = 40
ACC_ROWS = 10240
ROWS_PER_TILE = ACC_ROWS // NS


def _sc_aggregate(edges3, x, zeros_blk):
    mesh = plsc.VectorSubcoreMesh(core_axis_name="c", subcore_axis_name="s")

    @functools.partial(
        pl.kernel,
        out_type=jax.ShapeDtypeStruct((NC, ACC_ROWS, F), jnp.float32),
        mesh=mesh,
        scratch_types=[
            pltpu.VMEM((STAGE, 2, CHUNK), jnp.int32),
            pltpu.VMEM((NBUF, CHUNK, F), jnp.float32),
            pltpu.VMEM_SHARED((ACC_ROWS, F), jnp.float32),
        ] + [pltpu.SemaphoreType.DMA] * (2 * NBUF),
    )
    def agg_kernel(edges_hbm, x_hbm, zeros_hbm, out_hbm,
                   eidx_v, bufs, acc, *sems):
        gsems = sems[:NBUF]
        ssems = sems[NBUF:]
        cid = lax.axis_index("c")
        sid = lax.axis_index("s")
        wid = cid * NS + sid
        base = sid * ROWS_PER_TILE

        pltpu.sync_copy(zeros_hbm, bufs.at[0])
        for t in range(ROWS_PER_TILE // CHUNK):
            pltpu.sync_copy(bufs.at[0], acc.at[pl.ds(base + t * CHUNK, CHUNK)])
        REM = ROWS_PER_TILE % CHUNK
        if REM:
            pltpu.sync_copy(
                bufs.at[0, pl.ds(0, REM)],
                acc.at[pl.ds(base + ROWS_PER_TILE - REM, REM)])
        plsc.subcore_barrier()

        c0 = jnp.where(wid < W72, CH72 * wid,
                       W72 * CH72 + CH80 * (wid - W72))
        nch2 = jnp.where(wid < W72, CH72 - STAGE, CH80 - STAGE)

        def run_stage(row0, ngroups, last_idx):
            row0 = pl.multiple_of(row0, 8)
            pltpu.sync_copy(edges_hbm.at[pl.ds(row0, STAGE)], eidx_v)

            for b in range(NBUF):
                pltpu.async_copy(x_hbm.at[eidx_v.at[b, 0]], bufs.at[b],
                                 gsems[b])

            def group(g, carry):
                j0 = g * NBUF
                for b in range(NBUF):
                    pltpu.make_async_copy(
                        x_hbm.at[eidx_v.at[j0 + b, 0]], bufs.at[b],
                        gsems[b]).wait()
                    pltpu.sync_copy(bufs.at[b], acc.at[eidx_v.at[j0 + b, 1]],
                                    add=True)
                    nj = jnp.minimum(j0 + NBUF + b, last_idx)
                    pltpu.async_copy(
                        x_hbm.at[eidx_v.at[nj, 0]], bufs.at[b], gsems[b])
                return carry

            lax.fori_loop(0, ngroups, group, 0)
            for b in range(NBUF):
                pltpu.make_async_copy(
                    x_hbm.at[eidx_v.at[last_idx, 0]], bufs.at[b],
                    gsems[b]).wait()

        run_stage(c0, STAGE // NBUF, STAGE - 1)
        run_stage(c0 + STAGE, nch2 // NBUF, nch2 - 1)

        @pl.when(wid < NEXTRA)
        def _extra():
            r0 = W72 * CH72 + (NW - W72) * CH80
            pltpu.sync_copy(edges_hbm.at[pl.ds(r0, NEXTRA)],
                            eidx_v.at[pl.ds(0, NEXTRA)])
            pltpu.async_copy(
                x_hbm.at[eidx_v.at[wid, 0]], bufs.at[0], gsems[0]).wait()
            pltpu.sync_copy(bufs.at[0], acc.at[eidx_v.at[wid, 1]], add=True)

        plsc.subcore_barrier()

        for t in range(ROWS_PER_TILE // CHUNK):
            r0 = base + t * CHUNK
            pltpu.sync_copy(acc.at[pl.ds(r0, CHUNK)], bufs.at[0])
            pltpu.sync_copy(bufs.at[0], out_hbm.at[cid, pl.ds(r0, CHUNK)])
        if REM:
            r0 = base + ROWS_PER_TILE - REM
            pltpu.sync_copy(acc.at[pl.ds(r0, REM)], bufs.at[0, pl.ds(0, REM)])
            pltpu.sync_copy(bufs.at[0, pl.ds(0, REM)],
                            out_hbm.at[cid, pl.ds(r0, REM)])

    return agg_kernel(edges3, x, zeros_blk)


BN = 1000
NBLK = N // BN


def _dense_kernel(x_ref, p_ref, b_ref, w1a_ref, b1a_ref, w1b_ref, b1b_ref,
                  gamma_ref, beta_ref, wl1_ref, bl1_ref, wl2_ref, bl2_ref,
                  wm1_ref, bm1_ref, wm2_ref, bm2_ref, out_ref,
                  sum_s, sq_s, seg_s, cnt_s):
    i = pl.program_id(0)

    @pl.when(i == 0)
    def _init():
        sum_s[...] = jnp.zeros_like(sum_s)
        sq_s[...] = jnp.zeros_like(sq_s)
        seg_s[...] = jnp.zeros_like(seg_s)
        cnt_s[...] = jnp.zeros_like(cnt_s)

    h = x_ref[...] + p_ref[0] + p_ref[1]
    h = lax.dot_general(h, w1a_ref[...], (((1,), (0,)), ((), ())),
                        preferred_element_type=jnp.float32) + b1a_ref[...]
    h = jnp.maximum(h, 0.0)
    h = lax.dot_general(h, w1b_ref[...], (((1,), (0,)), ((), ())),
                        preferred_element_type=jnp.float32) + b1b_ref[...]
    x1 = jnp.maximum(h, 0.0)

    sum_s[...] += jnp.sum(x1, axis=0, keepdims=True)
    sq_s[...] += jnp.sum(x1 * x1, axis=0, keepdims=True)

    bb = b_ref[0]
    onehot = (bb.reshape(BN, 1) ==
              lax.broadcasted_iota(jnp.int32, (1, G), 1)).astype(jnp.float32)
    seg_s[...] += lax.dot_general(onehot, x1, (((0,), (0,)), ((), ())),
                                  preferred_element_type=jnp.float32)
    cnt_s[...] += jnp.sum(onehot, axis=0, keepdims=True)

    @pl.when(i == NBLK - 1)
    def _finish():
        mean = sum_s[...] / float(N)
        var = sq_s[...] / float(N) - mean * mean
        scale = gamma_ref[...] * lax.rsqrt(var + 1e-5)
        counts = cnt_s[...]
        counts_col = counts.reshape(G, 1)
        seg = seg_s[...]
        pooled = scale * (seg - counts_col * mean) + counts_col * beta_ref[...]
        x_new = pooled / jnp.maximum(counts_col, 1.0)

        def mm(a, b):
            return lax.dot_general(a, b, (((1,), (0,)), ((), ())),
                                   preferred_element_type=jnp.float32)

        wl1 = wl1_ref[...]
        wl2 = wl2_ref[...]
        base_p = mm(x_new, wl1[:F]) + bl1_ref[...]
        base_q = mm(x_new, wl2[:F]) + bl2_ref[...]
        hh = x_new
        for _ in range(ITERS):
            p = base_p + mm(hh, wl1[F:])
            q = base_q + mm(hh, wl2[F:])
            hh = jnp.concatenate(
                [jnp.tanh(q), 1.0 / (1.0 + jnp.exp(-p))], axis=1)

        o = jnp.maximum(mm(hh, wm1_ref[...]) + bm1_ref[...], 0.0)
        o = mm(o, wm2_ref[...]) + bm2_ref[...]
        out_ref[...] = o


def kernel(x, edge_index, batch, W1a, b1a, W1b, b1b, gamma, beta,
           Wl1, bl1, Wl2, bl2, Wm1, bm1, Wm2, bm2):
    edges3 = edge_index.reshape(2, NCHUNKS, CHUNK).transpose(1, 0, 2)
    zeros_blk = jnp.zeros((CHUNK, F), jnp.float32)

    partials = _sc_aggregate(edges3, x, zeros_blk)

    DIM = Wm1.shape[1]
    OUT = Wm2.shape[1]
    batch3 = batch.reshape(NBLK, 1, BN)

    row_spec = pl.BlockSpec((BN, F), lambda i: (i, 0))
    full = lambda shape: pl.BlockSpec(shape, lambda i: tuple(0 for _ in shape))

    out = pl.pallas_call(
        _dense_kernel,
        grid=(NBLK,),
        in_specs=[
            row_spec,
            pl.BlockSpec((NC, BN, F), lambda i: (0, i, 0)),
            pl.BlockSpec((1, 1, BN), lambda i: (i, 0, 0)),
            full((F, F)), full((1, F)), full((F, F)), full((1, F)),
            full((1, F)), full((1, F)),
            full((2 * F, G)), full((1, G)), full((2 * F, G)), full((1, G)),
            full((F, DIM)), full((1, DIM)), full((DIM, OUT)), full((1, OUT)),
        ],
        out_specs=pl.BlockSpec((G, OUT), lambda i: (0, 0)),
        out_shape=jax.ShapeDtypeStruct((G, OUT), jnp.float32),
        scratch_shapes=[
            pltpu.VMEM((1, F), jnp.float32),
            pltpu.VMEM((1, F), jnp.float32),
            pltpu.VMEM((G, F), jnp.float32),
            pltpu.VMEM((1, G), jnp.float32),
        ],
    )(x, partials, batch3, W1a, b1a.reshape(1, F), W1b, b1b.reshape(1, F),
      gamma.reshape(1, F), beta.reshape(1, F),
      Wl1, bl1.reshape(1, G), Wl2, bl2.reshape(1, G),
      Wm1, bm1.reshape(1, DIM), Wm2, bm2.reshape(1, OUT))

    return out


# speedup vs baseline: 1.2034x; 1.0231x over previous
import functools

import jax
import jax.numpy as jnp
from jax import lax
from jax.experimental import pallas as pl
from jax.experimental.pallas import tpu as pltpu
from jax.experimental.pallas import tpu_sc as plsc

N = 10000
E = 320000
F = 128
G = 64
ITERS = 10

NC = 2
NS = 16
NW = NC * NS
CHUNK = 128
NBUF = 2
NCHUNKS = E // CHUNK
CH80 = 80
CH72 = 72
W72 = 8
NEXTRA = NCHUNKS - W72 * CH72 - (NW - W72) * CH80
STAGE = 40
ACC_ROWS = 10240
ROWS_PER_TILE = ACC_ROWS // NS


def _sc_aggregate(edges3, x, zeros_blk):
    mesh = plsc.VectorSubcoreMesh(core_axis_name="c", subcore_axis_name="s")

    @functools.partial(
        pl.kernel,
        out_type=jax.ShapeDtypeStruct((NC, ACC_ROWS, F), jnp.float32),
        mesh=mesh,
        scratch_types=[
            pltpu.VMEM((STAGE, 2, CHUNK), jnp.int32),
            pltpu.VMEM((NBUF, CHUNK, F), jnp.float32),
            pltpu.VMEM_SHARED((ACC_ROWS, F), jnp.float32),
        ] + [pltpu.SemaphoreType.DMA] * (2 * NBUF),
    )
    def agg_kernel(edges_hbm, x_hbm, zeros_hbm, out_hbm,
                   eidx_v, bufs, acc, *sems):
        gsems = sems[:NBUF]
        ssems = sems[NBUF:]
        cid = lax.axis_index("c")
        sid = lax.axis_index("s")
        wid = cid * NS + sid
        base = sid * ROWS_PER_TILE

        pltpu.sync_copy(zeros_hbm, bufs.at[0])
        for t in range(ROWS_PER_TILE // CHUNK):
            pltpu.sync_copy(bufs.at[0], acc.at[pl.ds(base + t * CHUNK, CHUNK)])
        REM = ROWS_PER_TILE % CHUNK
        if REM:
            pltpu.sync_copy(
                bufs.at[0, pl.ds(0, REM)],
                acc.at[pl.ds(base + ROWS_PER_TILE - REM, REM)])
        plsc.subcore_barrier()

        c0 = jnp.where(wid < W72, CH72 * wid,
                       W72 * CH72 + CH80 * (wid - W72))
        nch2 = jnp.where(wid < W72, CH72 - STAGE, CH80 - STAGE)

        def run_stage(row0, ngroups, last_idx):
            row0 = pl.multiple_of(row0, 8)
            pltpu.sync_copy(edges_hbm.at[pl.ds(row0, STAGE)], eidx_v)

            for b in range(NBUF):
                pltpu.async_copy(x_hbm.at[eidx_v.at[b, 0]], bufs.at[b],
                                 gsems[b])

            def group(g, carry):
                j0 = g * NBUF
                for b in range(NBUF):
                    pltpu.make_async_copy(
                        x_hbm.at[eidx_v.at[j0 + b, 0]], bufs.at[b],
                        gsems[b]).wait()
                    pltpu.sync_copy(bufs.at[b], acc.at[eidx_v.at[j0 + b, 1]],
                                    add=True)
                    nj = jnp.minimum(j0 + NBUF + b, last_idx)
                    pltpu.async_copy(
                        x_hbm.at[eidx_v.at[nj, 0]], bufs.at[b], gsems[b])
                return carry

            lax.fori_loop(0, ngroups, group, 0)
            for b in range(NBUF):
                pltpu.make_async_copy(
                    x_hbm.at[eidx_v.at[last_idx, 0]], bufs.at[b],
                    gsems[b]).wait()

        run_stage(c0, STAGE // NBUF, STAGE - 1)
        run_stage(c0 + STAGE, nch2 // NBUF, nch2 - 1)

        @pl.when(wid < NEXTRA)
        def _extra():
            r0 = W72 * CH72 + (NW - W72) * CH80
            pltpu.sync_copy(edges_hbm.at[pl.ds(r0, NEXTRA)],
                            eidx_v.at[pl.ds(0, NEXTRA)])
            pltpu.async_copy(
                x_hbm.at[eidx_v.at[wid, 0]], bufs.at[0], gsems[0]).wait()
            pltpu.sync_copy(bufs.at[0], acc.at[eidx_v.at[wid, 1]], add=True)

        plsc.subcore_barrier()

        for t in range(ROWS_PER_TILE // CHUNK):
            r0 = base + t * CHUNK
            pltpu.sync_copy(acc.at[pl.ds(r0, CHUNK)], bufs.at[0])
            pltpu.sync_copy(bufs.at[0], out_hbm.at[cid, pl.ds(r0, CHUNK)])
        if REM:
            r0 = base + ROWS_PER_TILE - REM
            pltpu.sync_copy(acc.at[pl.ds(r0, REM)], bufs.at[0, pl.ds(0, REM)])
            pltpu.sync_copy(bufs.at[0, pl.ds(0, REM)],
                            out_hbm.at[cid, pl.ds(r0, REM)])

    return agg_kernel(edges3, x, zeros_blk)


BN = 2000
NBLK = N // BN


def _dense_kernel(x_ref, p_ref, b_ref, w1a_ref, b1a_ref, w1b_ref, b1b_ref,
                  gamma_ref, beta_ref, wl1_ref, bl1_ref, wl2_ref, bl2_ref,
                  wm1_ref, bm1_ref, wm2_ref, bm2_ref, out_ref,
                  sum_s, sq_s, seg_s, cnt_s):
    i = pl.program_id(0)

    @pl.when(i == 0)
    def _init():
        sum_s[...] = jnp.zeros_like(sum_s)
        sq_s[...] = jnp.zeros_like(sq_s)
        seg_s[...] = jnp.zeros_like(seg_s)
        cnt_s[...] = jnp.zeros_like(cnt_s)

    h = x_ref[...] + p_ref[0] + p_ref[1]
    h = lax.dot_general(h, w1a_ref[...], (((1,), (0,)), ((), ())),
                        preferred_element_type=jnp.float32) + b1a_ref[...]
    h = jnp.maximum(h, 0.0)
    h = lax.dot_general(h, w1b_ref[...], (((1,), (0,)), ((), ())),
                        preferred_element_type=jnp.float32) + b1b_ref[...]
    x1 = jnp.maximum(h, 0.0)

    sum_s[...] += jnp.sum(x1, axis=0, keepdims=True)
    sq_s[...] += jnp.sum(x1 * x1, axis=0, keepdims=True)

    bb = b_ref[0]
    onehot = (bb.reshape(BN, 1) ==
              lax.broadcasted_iota(jnp.int32, (1, G), 1)).astype(jnp.float32)
    seg_s[...] += lax.dot_general(onehot, x1, (((0,), (0,)), ((), ())),
                                  preferred_element_type=jnp.float32)
    cnt_s[...] += jnp.sum(onehot, axis=0, keepdims=True)

    @pl.when(i == NBLK - 1)
    def _finish():
        mean = sum_s[...] / float(N)
        var = sq_s[...] / float(N) - mean * mean
        scale = gamma_ref[...] * lax.rsqrt(var + 1e-5)
        counts = cnt_s[...]
        counts_col = counts.reshape(G, 1)
        seg = seg_s[...]
        pooled = scale * (seg - counts_col * mean) + counts_col * beta_ref[...]
        x_new = pooled / jnp.maximum(counts_col, 1.0)

        def mm(a, b):
            return lax.dot_general(a, b, (((1,), (0,)), ((), ())),
                                   preferred_element_type=jnp.float32)

        wl1 = wl1_ref[...]
        wl2 = wl2_ref[...]
        base_p = mm(x_new, wl1[:F]) + bl1_ref[...]
        base_q = mm(x_new, wl2[:F]) + bl2_ref[...]
        hh = x_new
        for _ in range(ITERS):
            p = base_p + mm(hh, wl1[F:])
            q = base_q + mm(hh, wl2[F:])
            hh = jnp.concatenate(
                [jnp.tanh(q), 1.0 / (1.0 + jnp.exp(-p))], axis=1)

        o = jnp.maximum(mm(hh, wm1_ref[...]) + bm1_ref[...], 0.0)
        o = mm(o, wm2_ref[...]) + bm2_ref[...]
        out_ref[...] = o


def kernel(x, edge_index, batch, W1a, b1a, W1b, b1b, gamma, beta,
           Wl1, bl1, Wl2, bl2, Wm1, bm1, Wm2, bm2):
    edges3 = edge_index.reshape(2, NCHUNKS, CHUNK).transpose(1, 0, 2)
    zeros_blk = jnp.zeros((CHUNK, F), jnp.float32)

    partials = _sc_aggregate(edges3, x, zeros_blk)

    DIM = Wm1.shape[1]
    OUT = Wm2.shape[1]
    batch3 = batch.reshape(NBLK, 1, BN)

    row_spec = pl.BlockSpec((BN, F), lambda i: (i, 0))
    full = lambda shape: pl.BlockSpec(shape, lambda i: tuple(0 for _ in shape))

    out = pl.pallas_call(
        _dense_kernel,
        grid=(NBLK,),
        in_specs=[
            row_spec,
            pl.BlockSpec((NC, BN, F), lambda i: (0, i, 0)),
            pl.BlockSpec((1, 1, BN), lambda i: (i, 0, 0)),
            full((F, F)), full((1, F)), full((F, F)), full((1, F)),
            full((1, F)), full((1, F)),
            full((2 * F, G)), full((1, G)), full((2 * F, G)), full((1, G)),
            full((F, DIM)), full((1, DIM)), full((DIM, OUT)), full((1, OUT)),
        ],
        out_specs=pl.BlockSpec((G, OUT), lambda i: (0, 0)),
        out_shape=jax.ShapeDtypeStruct((G, OUT), jnp.float32),
        scratch_shapes=[
            pltpu.VMEM((1, F), jnp.float32),
            pltpu.VMEM((1, F), jnp.float32),
            pltpu.VMEM((G, F), jnp.float32),
            pltpu.VMEM((1, G), jnp.float32),
        ],
    )(x, partials, batch3, W1a, b1a.reshape(1, F), W1b, b1b.reshape(1, F),
      gamma.reshape(1, F), beta.reshape(1, F),
      Wl1, bl1.reshape(1, G), Wl2, bl2.reshape(1, G),
      Wm1, bm1.reshape(1, DIM), Wm2, bm2.reshape(1, OUT))

    return out
